# baseline (speedup 1.0000x reference)
import functools

import jax
import jax.numpy as jnp
from jax import lax
from jax.experimental import pallas as pl
from jax.experimental.pallas import tpu as pltpu
from jax.experimental.pallas import tpu_sc as plsc

N_NODES = 10000
D = 128
N_EDGE_TYPES = 3
E_PER_TYPE = 213334
N_LAYERS = 3

NC = 2
NS = 16
N_TILES = NC * NS
CHUNK = 128
CHUNKS_PER_TILE = 54
CHUNK_STRIDE = 56
E_PAD = N_TILES * CHUNK * CHUNKS_PER_TILE
NBUF = 2
ACC_ROWS = 10240
DUMMY_DST = N_NODES
ROWS_PER_SUBCORE_ZERO = ACC_ROWS // NS
ROWS_PER_SUBCORE_OUT = N_NODES // NS

BR = 2000
NBLK = N_NODES // BR


def _msg_body(h_ref, w_ref, b_ref, out_ref):
    out_ref[0] = (
        jnp.dot(h_ref[...], w_ref[0], preferred_element_type=jnp.float32)
        + b_ref[0]
    )


def _msg_matmul(h, Wm, bm):
    return pl.pallas_call(
        _msg_body,
        grid=(NBLK, N_EDGE_TYPES),
        in_specs=[
            pl.BlockSpec((BR, D), lambda i, t: (i, 0)),
            pl.BlockSpec((1, D, D), lambda i, t: (t, 0, 0)),
            pl.BlockSpec((1, 1, D), lambda i, t: (t, 0, 0)),
        ],
        out_specs=pl.BlockSpec((1, BR, D), lambda i, t: (t, i, 0)),
        out_shape=jax.ShapeDtypeStruct((N_EDGE_TYPES, N_NODES, D), jnp.float32),
    )(h, Wm, bm.reshape(N_EDGE_TYPES, 1, D))


def _u_body(h_ref, ug_ref, bg_ref, out_ref):
    for i in range(2):
        out_ref[i] = (
            jnp.dot(h_ref[...], ug_ref[i], preferred_element_type=jnp.float32)
            + bg_ref[i]
        )


def _u_terms(h, Ug, bg):
    return pl.pallas_call(
        _u_body,
        grid=(NBLK,),
        in_specs=[
            pl.BlockSpec((BR, D), lambda i: (i, 0)),
            pl.BlockSpec((2, D, D), lambda i: (0, 0, 0)),
            pl.BlockSpec((2, 1, D), lambda i: (0, 0, 0)),
        ],
        out_specs=pl.BlockSpec((2, BR, D), lambda i: (0, i, 0)),
        out_shape=jax.ShapeDtypeStruct((2, N_NODES, D), jnp.float32),
    )(h, Ug[:2], bg[:2].reshape(2, 1, D))


def _gru_math(p_ref, h_ref, u_ref, wg_ref, ug_ref, bg_ref):
    a = p_ref[0] + p_ref[1]
    h = h_ref[...]
    dot = functools.partial(jnp.dot, preferred_element_type=jnp.float32)
    z = jax.nn.sigmoid(dot(a, wg_ref[0]) + u_ref[0])
    r = jax.nn.sigmoid(dot(a, wg_ref[1]) + u_ref[1])
    ht = jnp.tanh(dot(a, wg_ref[2]) + dot(r * h, ug_ref[2]) + bg_ref[2])
    return (1.0 - z) * h + z * ht


def _gru_body(p_ref, h_ref, u_ref, wg_ref, ug_ref, bg_ref, out_ref):
    out_ref[...] = _gru_math(p_ref, h_ref, u_ref, wg_ref, ug_ref, bg_ref)


def _gru_msg_body(p_ref, h_ref, u_ref, wg_ref, ug_ref, bg_ref, wm_ref,
                  bm_ref, h_out_ref, xw_out_ref):
    hn = _gru_math(p_ref, h_ref, u_ref, wg_ref, ug_ref, bg_ref)
    h_out_ref[...] = hn
    for t in range(N_EDGE_TYPES):
        xw_out_ref[t] = (
            jnp.dot(hn, wm_ref[t], preferred_element_type=jnp.float32)
            + bm_ref[t]
        )


def _gru_msg(parts, h, u, Wg, Ug, bg, Wm, bm):
    return pl.pallas_call(
        _gru_msg_body,
        grid=(NBLK,),
        in_specs=[
            pl.BlockSpec((NC, BR, D), lambda i: (0, i, 0)),
            pl.BlockSpec((BR, D), lambda i: (i, 0)),
            pl.BlockSpec((2, BR, D), lambda i: (0, i, 0)),
            pl.BlockSpec((3, D, D), lambda i: (0, 0, 0)),
            pl.BlockSpec((3, D, D), lambda i: (0, 0, 0)),
            pl.BlockSpec((3, 1, D), lambda i: (0, 0, 0)),
            pl.BlockSpec((N_EDGE_TYPES, D, D), lambda i: (0, 0, 0)),
            pl.BlockSpec((N_EDGE_TYPES, 1, D), lambda i: (0, 0, 0)),
        ],
        out_specs=[
            pl.BlockSpec((BR, D), lambda i: (i, 0)),
            pl.BlockSpec((N_EDGE_TYPES, BR, D), lambda i: (0, i, 0)),
        ],
        out_shape=[
            jax.ShapeDtypeStruct((N_NODES, D), jnp.float32),
            jax.ShapeDtypeStruct((N_EDGE_TYPES, N_NODES, D), jnp.float32),
        ],
    )(parts, h, u, Wg, Ug, bg.reshape(3, 1, D), Wm,
      bm.reshape(N_EDGE_TYPES, 1, D))


def _gru_update(parts, h, u, Wg, Ug, bg):
    return pl.pallas_call(
        _gru_body,
        grid=(NBLK,),
        in_specs=[
            pl.BlockSpec((NC, BR, D), lambda i: (0, i, 0)),
            pl.BlockSpec((BR, D), lambda i: (i, 0)),
            pl.BlockSpec((2, BR, D), lambda i: (0, i, 0)),
            pl.BlockSpec((3, D, D), lambda i: (0, 0, 0)),
            pl.BlockSpec((3, D, D), lambda i: (0, 0, 0)),
            pl.BlockSpec((3, 1, D), lambda i: (0, 0, 0)),
        ],
        out_specs=pl.BlockSpec((BR, D), lambda i: (i, 0)),
        out_shape=jax.ShapeDtypeStruct((N_NODES, D), jnp.float32),
    )(parts, h, u, Wg, Ug, bg.reshape(3, 1, D))


def _edge_pass_body(xw_hbm, src_hbm, dst_hbm, out_hbm,
                    acc, src_v, dst_v, rows, sems):
    cid = lax.axis_index("c")
    sid = lax.axis_index("s")
    tile = cid * NS + sid
    cpt = CHUNKS_PER_TILE

    @pl.loop(0, CHUNK)
    def _(i):
        @pl.loop(0, D, step=16)
        def _(j):
            rows[0][i, pl.ds(j, 16)] = jnp.zeros((16,), jnp.float32)

    zcopy = 64
    zbase = sid * ROWS_PER_SUBCORE_ZERO
    @pl.loop(0, ROWS_PER_SUBCORE_ZERO // zcopy)
    def _(k):
        pltpu.sync_copy(rows[0].at[pl.ds(0, zcopy)],
                        acc.at[pl.ds(zbase + k * zcopy, zcopy)])

    plsc.subcore_barrier()

    for t in range(N_EDGE_TYPES):
        rbase = (t * N_TILES + tile) * CHUNK_STRIDE
        pltpu.sync_copy(src_hbm.at[pl.ds(rbase, CHUNK_STRIDE)], src_v)
        pltpu.sync_copy(dst_hbm.at[pl.ds(rbase, CHUNK_STRIDE)], dst_v)
        table = xw_hbm.at[t]

        for b in range(NBUF - 1):
            pltpu.async_copy(table.at[src_v.at[b]], rows[b], sems[b])

        @pl.loop(0, cpt, step=NBUF)
        def _(c):
            for j in range(NBUF):
                nxt = lax.rem(c + j + NBUF - 1, cpt)
                b = (j + NBUF - 1) % NBUF
                pltpu.async_copy(table.at[src_v.at[nxt]], rows[b], sems[b])
                pltpu.make_async_copy(table.at[src_v.at[c + j]],
                                      rows[j], sems[j]).wait()
                pltpu.sync_copy(rows[j], acc.at[dst_v.at[c + j]], add=True)

        for b in range(NBUF - 1):
            pltpu.make_async_copy(table.at[src_v.at[b]], rows[b],
                                  sems[b]).wait()

    plsc.subcore_barrier()

    obase = sid * ROWS_PER_SUBCORE_ZERO
    pltpu.sync_copy(acc.at[pl.ds(obase, ROWS_PER_SUBCORE_ZERO)],
                    out_hbm.at[cid].at[pl.ds(obase, ROWS_PER_SUBCORE_ZERO)])


def _edge_pass(xw, src, dst):
    mesh = plsc.VectorSubcoreMesh(core_axis_name="c", subcore_axis_name="s")
    k = pl.kernel(
        _edge_pass_body,
        out_type=jax.ShapeDtypeStruct((NC, ACC_ROWS, D), jnp.float32),
        mesh=mesh,
        scratch_types=[
            pltpu.VMEM_SHARED((ACC_ROWS, D), jnp.float32),
            pltpu.VMEM((CHUNK_STRIDE, CHUNK), jnp.int32),
            pltpu.VMEM((CHUNK_STRIDE, CHUNK), jnp.int32),
            [pltpu.VMEM((CHUNK, D), jnp.float32) for _ in range(NBUF)],
            [pltpu.SemaphoreType.DMA for _ in range(NBUF)],
        ],
    )
    return k(xw, src, dst)


def kernel(x, x_lengths, edge_list, W_msg, b_msg, W_gru, U_gru, b_gru):
    del x_lengths
    src = edge_list[:, 0, :]
    dst = edge_list[:, 1, :]
    pad = E_PAD - E_PER_TYPE
    dummy_dst = DUMMY_DST + (jnp.arange(pad, dtype=jnp.int32)
                             % (ACC_ROWS - N_NODES))
    dummy_src = jnp.arange(pad, dtype=jnp.int32) % N_NODES
    src = jnp.concatenate(
        [src, jnp.broadcast_to(dummy_src, (N_EDGE_TYPES, pad))], axis=1)
    dst = jnp.concatenate(
        [dst, jnp.broadcast_to(dummy_dst, (N_EDGE_TYPES, pad))], axis=1)
    src = src.reshape(N_EDGE_TYPES, N_TILES, CHUNKS_PER_TILE, CHUNK)
    dst = dst.reshape(N_EDGE_TYPES, N_TILES, CHUNKS_PER_TILE, CHUNK)
    blk_pad = ((0, 0), (0, 0), (0, CHUNK_STRIDE - CHUNKS_PER_TILE), (0, 0))
    src = jnp.pad(src, blk_pad).reshape(-1, CHUNK)
    dst = jnp.pad(dst, blk_pad, constant_values=DUMMY_DST).reshape(-1, CHUNK)

    h = x
    xw = _msg_matmul(h, W_msg[0], b_msg[0])
    u = _u_terms(h, U_gru[0], b_gru[0])
    parts = _edge_pass(xw, src, dst)
    for l in range(1, N_LAYERS):
        h, xw = _gru_msg(parts, h, u, W_gru[l - 1], U_gru[l - 1],
                         b_gru[l - 1], W_msg[l], b_msg[l])
        u = _u_terms(h, U_gru[l], b_gru[l])
        parts = _edge_pass(xw, src, dst)
    return _gru_update(parts, h, u, W_gru[N_LAYERS - 1], U_gru[N_LAYERS - 1],
                       b_gru[N_LAYERS - 1])


# speedup vs baseline: 1.0198x; 1.0198x over previous
import functools

import jax
import jax.numpy as jnp
from jax import lax
from jax.experimental import pallas as pl
from jax.experimental.pallas import tpu as pltpu
from jax.experimental.pallas import tpu_sc as plsc

N_NODES = 10000
D = 128
N_EDGE_TYPES = 3
E_PER_TYPE = 213334
N_LAYERS = 3

NC = 2
NS = 16
N_TILES = NC * NS
CHUNK = 128
CHUNKS_PER_TILE = 54
CHUNK_STRIDE = 56
E_PAD = N_TILES * CHUNK * CHUNKS_PER_TILE
NBUF = 2
ACC_ROWS = 10240
DUMMY_DST = N_NODES
ROWS_PER_SUBCORE_ZERO = ACC_ROWS // NS
ROWS_PER_SUBCORE_OUT = N_NODES // NS

BR = 2000
NBLK = N_NODES // BR


def _msg_body(h_ref, w_ref, b_ref, out_ref):
    out_ref[0] = (
        jnp.dot(h_ref[...], w_ref[0], preferred_element_type=jnp.float32)
        + b_ref[0]
    )


def _msg_matmul(h, Wm, bm):
    return pl.pallas_call(
        _msg_body,
        grid=(NBLK, N_EDGE_TYPES),
        in_specs=[
            pl.BlockSpec((BR, D), lambda i, t: (i, 0)),
            pl.BlockSpec((1, D, D), lambda i, t: (t, 0, 0)),
            pl.BlockSpec((1, 1, D), lambda i, t: (t, 0, 0)),
        ],
        out_specs=pl.BlockSpec((1, BR, D), lambda i, t: (t, i, 0)),
        out_shape=jax.ShapeDtypeStruct((N_EDGE_TYPES, N_NODES, D), jnp.float32),
    )(h, Wm, bm.reshape(N_EDGE_TYPES, 1, D))


def _gru_math(p_ref, h_ref, wg_ref, ug_ref, bg_ref):
    a = p_ref[0] + p_ref[1]
    h = h_ref[...]
    dot = functools.partial(jnp.dot, preferred_element_type=jnp.float32)
    z = jax.nn.sigmoid(dot(a, wg_ref[0]) + dot(h, ug_ref[0]) + bg_ref[0])
    r = jax.nn.sigmoid(dot(a, wg_ref[1]) + dot(h, ug_ref[1]) + bg_ref[1])
    ht = jnp.tanh(dot(a, wg_ref[2]) + dot(r * h, ug_ref[2]) + bg_ref[2])
    return (1.0 - z) * h + z * ht


def _gru_body(p_ref, h_ref, wg_ref, ug_ref, bg_ref, out_ref):
    out_ref[...] = _gru_math(p_ref, h_ref, wg_ref, ug_ref, bg_ref)


def _gru_msg_body(p_ref, h_ref, wg_ref, ug_ref, bg_ref, wm_ref,
                  bm_ref, h_out_ref, xw_out_ref):
    hn = _gru_math(p_ref, h_ref, wg_ref, ug_ref, bg_ref)
    h_out_ref[...] = hn
    for t in range(N_EDGE_TYPES):
        xw_out_ref[t] = (
            jnp.dot(hn, wm_ref[t], preferred_element_type=jnp.float32)
            + bm_ref[t]
        )


def _gru_msg(parts, h, Wg, Ug, bg, Wm, bm):
    return pl.pallas_call(
        _gru_msg_body,
        grid=(NBLK,),
        in_specs=[
            pl.BlockSpec((NC, BR, D), lambda i: (0, i, 0)),
            pl.BlockSpec((BR, D), lambda i: (i, 0)),
            pl.BlockSpec((3, D, D), lambda i: (0, 0, 0)),
            pl.BlockSpec((3, D, D), lambda i: (0, 0, 0)),
            pl.BlockSpec((3, 1, D), lambda i: (0, 0, 0)),
            pl.BlockSpec((N_EDGE_TYPES, D, D), lambda i: (0, 0, 0)),
            pl.BlockSpec((N_EDGE_TYPES, 1, D), lambda i: (0, 0, 0)),
        ],
        out_specs=[
            pl.BlockSpec((BR, D), lambda i: (i, 0)),
            pl.BlockSpec((N_EDGE_TYPES, BR, D), lambda i: (0, i, 0)),
        ],
        out_shape=[
            jax.ShapeDtypeStruct((N_NODES, D), jnp.float32),
            jax.ShapeDtypeStruct((N_EDGE_TYPES, N_NODES, D), jnp.float32),
        ],
    )(parts, h, Wg, Ug, bg.reshape(3, 1, D), Wm,
      bm.reshape(N_EDGE_TYPES, 1, D))


def _gru_update(parts, h, Wg, Ug, bg):
    return pl.pallas_call(
        _gru_body,
        grid=(NBLK,),
        in_specs=[
            pl.BlockSpec((NC, BR, D), lambda i: (0, i, 0)),
            pl.BlockSpec((BR, D), lambda i: (i, 0)),
            pl.BlockSpec((3, D, D), lambda i: (0, 0, 0)),
            pl.BlockSpec((3, D, D), lambda i: (0, 0, 0)),
            pl.BlockSpec((3, 1, D), lambda i: (0, 0, 0)),
        ],
        out_specs=pl.BlockSpec((BR, D), lambda i: (i, 0)),
        out_shape=jax.ShapeDtypeStruct((N_NODES, D), jnp.float32),
    )(parts, h, Wg, Ug, bg.reshape(3, 1, D))


def _edge_pass_body(xw_hbm, src_hbm, dst_hbm, out_hbm,
                    acc, src_v, dst_v, rows, sems):
    cid = lax.axis_index("c")
    sid = lax.axis_index("s")
    tile = cid * NS + sid
    cpt = CHUNKS_PER_TILE

    @pl.loop(0, CHUNK)
    def _(i):
        @pl.loop(0, D, step=16)
        def _(j):
            rows[0][i, pl.ds(j, 16)] = jnp.zeros((16,), jnp.float32)

    zcopy = 64
    zbase = sid * ROWS_PER_SUBCORE_ZERO
    @pl.loop(0, ROWS_PER_SUBCORE_ZERO // zcopy)
    def _(k):
        pltpu.sync_copy(rows[0].at[pl.ds(0, zcopy)],
                        acc.at[pl.ds(zbase + k * zcopy, zcopy)])

    plsc.subcore_barrier()

    for t in range(N_EDGE_TYPES):
        rbase = (t * N_TILES + tile) * CHUNK_STRIDE
        pltpu.sync_copy(src_hbm.at[pl.ds(rbase, CHUNK_STRIDE)], src_v)
        pltpu.sync_copy(dst_hbm.at[pl.ds(rbase, CHUNK_STRIDE)], dst_v)
        table = xw_hbm.at[t]

        for b in range(NBUF - 1):
            pltpu.async_copy(table.at[src_v.at[b]], rows[b], sems[b])

        @pl.loop(0, cpt, step=NBUF)
        def _(c):
            for j in range(NBUF):
                nxt = lax.rem(c + j + NBUF - 1, cpt)
                b = (j + NBUF - 1) % NBUF
                pltpu.async_copy(table.at[src_v.at[nxt]], rows[b], sems[b])
                pltpu.make_async_copy(table.at[src_v.at[c + j]],
                                      rows[j], sems[j]).wait()
                pltpu.sync_copy(rows[j], acc.at[dst_v.at[c + j]], add=True)

        for b in range(NBUF - 1):
            pltpu.make_async_copy(table.at[src_v.at[b]], rows[b],
                                  sems[b]).wait()

    plsc.subcore_barrier()

    obase = sid * ROWS_PER_SUBCORE_ZERO
    pltpu.sync_copy(acc.at[pl.ds(obase, ROWS_PER_SUBCORE_ZERO)],
                    out_hbm.at[cid].at[pl.ds(obase, ROWS_PER_SUBCORE_ZERO)])


def _edge_pass(xw, src, dst):
    mesh = plsc.VectorSubcoreMesh(core_axis_name="c", subcore_axis_name="s")
    k = pl.kernel(
        _edge_pass_body,
        out_type=jax.ShapeDtypeStruct((NC, ACC_ROWS, D), jnp.float32),
        mesh=mesh,
        scratch_types=[
            pltpu.VMEM_SHARED((ACC_ROWS, D), jnp.float32),
            pltpu.VMEM((CHUNK_STRIDE, CHUNK), jnp.int32),
            pltpu.VMEM((CHUNK_STRIDE, CHUNK), jnp.int32),
            [pltpu.VMEM((CHUNK, D), jnp.float32) for _ in range(NBUF)],
            [pltpu.SemaphoreType.DMA for _ in range(NBUF)],
        ],
    )
    return k(xw, src, dst)


def kernel(x, x_lengths, edge_list, W_msg, b_msg, W_gru, U_gru, b_gru):
    del x_lengths
    src = edge_list[:, 0, :]
    dst = edge_list[:, 1, :]
    pad = E_PAD - E_PER_TYPE
    dummy_dst = DUMMY_DST + (jnp.arange(pad, dtype=jnp.int32)
                             % (ACC_ROWS - N_NODES))
    dummy_src = jnp.arange(pad, dtype=jnp.int32) % N_NODES
    src = jnp.concatenate(
        [src, jnp.broadcast_to(dummy_src, (N_EDGE_TYPES, pad))], axis=1)
    dst = jnp.concatenate(
        [dst, jnp.broadcast_to(dummy_dst, (N_EDGE_TYPES, pad))], axis=1)
    src = src.reshape(N_EDGE_TYPES, N_TILES, CHUNKS_PER_TILE, CHUNK)
    dst = dst.reshape(N_EDGE_TYPES, N_TILES, CHUNKS_PER_TILE, CHUNK)
    blk_pad = ((0, 0), (0, 0), (0, CHUNK_STRIDE - CHUNKS_PER_TILE), (0, 0))
    src = jnp.pad(src, blk_pad).reshape(-1, CHUNK)
    dst = jnp.pad(dst, blk_pad, constant_values=DUMMY_DST).reshape(-1, CHUNK)

    h = x
    xw = _msg_matmul(h, W_msg[0], b_msg[0])
    parts = _edge_pass(xw, src, dst)
    for l in range(1, N_LAYERS):
        h, xw = _gru_msg(parts, h, W_gru[l - 1], U_gru[l - 1],
                         b_gru[l - 1], W_msg[l], b_msg[l])
        parts = _edge_pass(xw, src, dst)
    return _gru_update(parts, h, W_gru[N_LAYERS - 1], U_gru[N_LAYERS - 1],
                       b_gru[N_LAYERS - 1])
